# baseline (speedup 1.0000x reference)
import functools

import jax
import jax.numpy as jnp
from jax import lax
from jax.experimental import pallas as pl
from jax.experimental.pallas import tpu as pltpu
from jax.experimental.pallas import tpu_sc as plsc

_BINS = 16
_LANES = 16
_ROWS = 96
_IMG = 512
_PIX = _IMG * _IMG
_CHROWS = 64
_CHUNK = _CHROWS * _IMG
_NCHUNK = _PIX // _CHUNK
_UNROLL = 8
_VREGS_PER_CHUNK = _CHUNK // _LANES

_INFO = plsc.get_sparse_core_info()
_NC = _INFO.num_cores
_NS = _INFO.num_subcores
_NW = _NC * _NS
_ROWS_PER_W = _ROWS // _NW


def _sc_hist(x_flat):
    mesh = plsc.VectorSubcoreMesh(core_axis_name="c", subcore_axis_name="s")

    @functools.partial(
        pl.kernel,
        out_type=jax.ShapeDtypeStruct((_NW, _ROWS_PER_W, _BINS), jnp.float32),
        mesh=mesh,
        compiler_params=pltpu.CompilerParams(needs_layout_passes=False),
        scratch_types=[
            pltpu.VMEM((2, _CHROWS, _IMG), jnp.float32),
            pltpu.VMEM((_BINS * _LANES,), jnp.float32),
            pltpu.VMEM((_ROWS_PER_W, _BINS), jnp.float32),
            pltpu.SemaphoreType.DMA,
            pltpu.SemaphoreType.DMA,
        ],
    )
    def body(x_hbm, out_hbm, buf, hist, outv, sem0, sem1):
        wid = lax.axis_index("s") * _NC + lax.axis_index("c")
        row0 = wid * _ROWS_PER_W
        sems = (sem0, sem1)
        lane = lax.iota(jnp.int32, 16)
        ones = jnp.ones((16,), jnp.float32)
        zeros = jnp.zeros((16,), jnp.float32)

        tasks = [(r, c) for r in range(_ROWS_PER_W) for c in range(_NCHUNK)]

        def start(i):
            r, c = tasks[i]
            return pltpu.async_copy(
                x_hbm.at[row0 + r, pl.ds(c * _CHROWS, _CHROWS), :],
                buf.at[i % 2],
                sems[i % 2],
            )

        copies = [start(0)]
        for i, (r, c) in enumerate(tasks):
            if i + 1 < len(tasks):
                copies.append(start(i + 1))
            if c == 0:
                for bb in range(_BINS):
                    hist[pl.ds(bb * _LANES, _LANES)] = zeros
            copies[i].wait()
            bref = buf.at[i % 2]

            @plsc.parallel_loop(0, _VREGS_PER_CHUNK, step=1, unroll=_UNROLL)
            def _(j):
                rr = lax.shift_right_logical(j, 5)
                cc = lax.shift_left(jnp.bitwise_and(j, 31), 4)
                v = bref[rr, pl.ds(cc, _LANES)]
                bits = plsc.bitcast(v + 1.0, jnp.int32)
                b16x = jnp.bitwise_and(
                    lax.shift_right_logical(bits, 15), 0xF0)
                idx = b16x + lane
                plsc.addupdate_scatter(hist, [idx], ones)

            if c == _NCHUNK - 1:
                tot = zeros
                lane16 = lane * 16
                for l in range(_LANES):
                    tot = tot + plsc.load_gather(hist, [lane16 + l])
                outv[r] = tot * (1.0 / _PIX)

        pltpu.sync_copy(outv, out_hbm.at[wid])

    return body(x_flat)


def _tc_hist(xr):
    k = xr.shape[0]

    def hbody(x_ref, o_ref):
        v = x_ref[0]
        bits = (v + 1.0).view(jnp.int32)
        bn = jnp.bitwise_and(lax.shift_right_logical(bits, 19), 0xF)
        cols = [jnp.sum((bn == i).astype(jnp.float32)) for i in range(_BINS)]
        row = jnp.concatenate(
            [jnp.stack(cols), jnp.zeros((128 - _BINS,), jnp.float32)])
        o_ref[...] = jnp.broadcast_to(row.reshape(1, 1, 128), (1, 8, 128))

    return pl.pallas_call(
        hbody,
        grid=(k,),
        in_specs=[pl.BlockSpec((1, _IMG, _IMG), lambda i: (i, 0, 0))],
        out_specs=pl.BlockSpec((1, 8, 128), lambda i: (i, 0, 0)),
        out_shape=jax.ShapeDtypeStruct((k, 8, 128), jnp.float32),
    )(xr)


def _fc(h, W, b):
    def fc_body(h_ref, w_ref, b_ref, o_ref):
        acc = jnp.dot(h_ref[...], w_ref[...],
                      preferred_element_type=jnp.float32)
        o_ref[...] = jnp.maximum(acc + b_ref[...], 0.0)

    return pl.pallas_call(
        fc_body,
        out_shape=jax.ShapeDtypeStruct((32, 64), jnp.float32),
    )(h, W, b.reshape(1, 64))


def kernel(x, W, b):
    x_flat = x.reshape(_ROWS, _IMG, _IMG)
    counts = _sc_hist(x_flat)
    tc_counts = _tc_hist(x_flat[_ROWS - 16:])
    h = counts.reshape(32, 48) + 0.0 * tc_counts[0, 0, 0]
    return _fc(h, W, b)


# speedup vs baseline: 1.0527x; 1.0527x over previous
import functools

import jax
import jax.numpy as jnp
from jax import lax
from jax.experimental import pallas as pl
from jax.experimental.pallas import tpu as pltpu
from jax.experimental.pallas import tpu_sc as plsc

_BINS = 16
_LANES = 16
_ROWS = 96
_IMG = 512
_PIX = _IMG * _IMG
_CHROWS = 64
_CHUNK = _CHROWS * _IMG
_NCHUNK = _PIX // _CHUNK
_UNROLL = 8
_VREGS_PER_CHUNK = _CHUNK // _LANES

_INFO = plsc.get_sparse_core_info()
_NC = _INFO.num_cores
_NS = _INFO.num_subcores
_NW = _NC * _NS
_ROWS_SC = 64
_ROWS_TC = _ROWS - _ROWS_SC
_ROWS_PER_W = _ROWS_SC // _NW


def _sc_hist(x_flat):
    mesh = plsc.VectorSubcoreMesh(core_axis_name="c", subcore_axis_name="s")

    @functools.partial(
        pl.kernel,
        out_type=jax.ShapeDtypeStruct((_NW, _ROWS_PER_W, _BINS), jnp.float32),
        mesh=mesh,
        compiler_params=pltpu.CompilerParams(needs_layout_passes=False),
        scratch_types=[
            pltpu.VMEM((2, _CHROWS, _IMG), jnp.float32),
            pltpu.VMEM((_BINS * _LANES,), jnp.float32),
            pltpu.VMEM((_ROWS_PER_W, _BINS), jnp.float32),
            pltpu.SemaphoreType.DMA,
            pltpu.SemaphoreType.DMA,
        ],
    )
    def body(x_hbm, out_hbm, buf, hist, outv, sem0, sem1):
        wid = lax.axis_index("s") * _NC + lax.axis_index("c")
        row0 = wid * _ROWS_PER_W
        sems = (sem0, sem1)
        lane = lax.iota(jnp.int32, 16)
        ones = jnp.ones((16,), jnp.float32)
        zeros = jnp.zeros((16,), jnp.float32)

        tasks = [(r, c) for r in range(_ROWS_PER_W) for c in range(_NCHUNK)]

        def start(i):
            r, c = tasks[i]
            return pltpu.async_copy(
                x_hbm.at[row0 + r, pl.ds(c * _CHROWS, _CHROWS), :],
                buf.at[i % 2],
                sems[i % 2],
            )

        copies = [start(0)]
        for i, (r, c) in enumerate(tasks):
            if i + 1 < len(tasks):
                copies.append(start(i + 1))
            if c == 0:
                for bb in range(_BINS):
                    hist[pl.ds(bb * _LANES, _LANES)] = zeros
            copies[i].wait()
            bref = buf.at[i % 2]

            @plsc.parallel_loop(0, _VREGS_PER_CHUNK, step=1, unroll=_UNROLL)
            def _(j):
                rr = lax.shift_right_logical(j, 5)
                cc = lax.shift_left(jnp.bitwise_and(j, 31), 4)
                v = bref[rr, pl.ds(cc, _LANES)]
                bits = plsc.bitcast(v + 1.0, jnp.int32)
                b16x = jnp.bitwise_and(
                    lax.shift_right_logical(bits, 15), 0xF0)
                idx = b16x + lane
                plsc.addupdate_scatter(hist, [idx], ones)

            if c == _NCHUNK - 1:
                tot = zeros
                lane16 = lane * 16
                for l in range(_LANES):
                    tot = tot + plsc.load_gather(hist, [lane16 + l])
                outv[r] = tot * (1.0 / _PIX)

        pltpu.sync_copy(outv, out_hbm.at[wid])

    return body(x_flat)


def _tc_hist(xr):
    k = _ROWS_TC

    def hbody(x_ref, o_ref):
        v = x_ref[0]
        bits = (v + 1.0).view(jnp.int32)
        bn = jnp.bitwise_and(lax.shift_right_logical(bits, 19), 0xF)
        cols = [jnp.sum((bn == i).astype(jnp.float32)) for i in range(_BINS)]
        row = jnp.concatenate(
            [jnp.stack(cols) * (1.0 / _PIX),
             jnp.zeros((128 - _BINS,), jnp.float32)])
        o_ref[...] = jnp.broadcast_to(row.reshape(1, 1, 128), (1, 8, 128))

    return pl.pallas_call(
        hbody,
        grid=(k,),
        in_specs=[pl.BlockSpec((1, _IMG, _IMG),
                               lambda i: (i + _ROWS_SC, 0, 0))],
        out_specs=pl.BlockSpec((1, 8, 128), lambda i: (i, 0, 0)),
        out_shape=jax.ShapeDtypeStruct((k, 8, 128), jnp.float32),
    )(xr)


def _fc(h, W, b):
    def fc_body(h_ref, w_ref, b_ref, o_ref):
        acc = jnp.dot(h_ref[...], w_ref[...],
                      preferred_element_type=jnp.float32)
        o_ref[...] = jnp.maximum(acc + b_ref[...], 0.0)

    return pl.pallas_call(
        fc_body,
        out_shape=jax.ShapeDtypeStruct((32, 64), jnp.float32),
    )(h, W, b.reshape(1, 64))


def kernel(x, W, b):
    x_flat = x.reshape(_ROWS, _IMG, _IMG)
    sc_means = _sc_hist(x_flat)
    tc_out = _tc_hist(x_flat)
    h = jnp.concatenate(
        [sc_means.reshape(_ROWS_SC, _BINS), tc_out[:, 0, :_BINS]],
        axis=0).reshape(32, 48)
    return _fc(h, W, b)


# speedup vs baseline: 1.3477x; 1.2803x over previous
import functools

import jax
import jax.numpy as jnp
from jax import lax
from jax.experimental import pallas as pl
from jax.experimental.pallas import tpu as pltpu
from jax.experimental.pallas import tpu_sc as plsc

_BINS = 16
_LANES = 16
_ROWS = 96
_IMG = 512
_PIX = _IMG * _IMG
_CHROWS = 64
_CHUNK = _CHROWS * _IMG
_NCHUNK = _PIX // _CHUNK
_UNROLL = 8
_VREGS_PER_CHUNK = _CHUNK // _LANES

_INFO = plsc.get_sparse_core_info()
_NC = _INFO.num_cores
_NS = _INFO.num_subcores
_NW = _NC * _NS
_ROWS_SC = 64
_ROWS_TC = _ROWS - _ROWS_SC
_ROWS_PER_W = _ROWS_SC // _NW


def _sc_hist(x_flat):
    mesh = plsc.VectorSubcoreMesh(core_axis_name="c", subcore_axis_name="s")

    @functools.partial(
        pl.kernel,
        out_type=jax.ShapeDtypeStruct((_NW, _ROWS_PER_W, _BINS), jnp.float32),
        mesh=mesh,
        compiler_params=pltpu.CompilerParams(needs_layout_passes=False),
        scratch_types=[
            pltpu.VMEM((2, _CHROWS, _IMG), jnp.float32),
            pltpu.VMEM((_BINS * _LANES,), jnp.float32),
            pltpu.VMEM((_ROWS_PER_W, _BINS), jnp.float32),
            pltpu.SemaphoreType.DMA,
            pltpu.SemaphoreType.DMA,
        ],
    )
    def body(x_hbm, out_hbm, buf, hist, outv, sem0, sem1):
        wid = lax.axis_index("s") * _NC + lax.axis_index("c")
        row0 = wid * _ROWS_PER_W
        sems = (sem0, sem1)
        lane = lax.iota(jnp.int32, 16)
        ones = jnp.ones((16,), jnp.float32)
        zeros = jnp.zeros((16,), jnp.float32)

        tasks = [(r, c) for r in range(_ROWS_PER_W) for c in range(_NCHUNK)]

        def start(i):
            r, c = tasks[i]
            return pltpu.async_copy(
                x_hbm.at[row0 + r, pl.ds(c * _CHROWS, _CHROWS), :],
                buf.at[i % 2],
                sems[i % 2],
            )

        copies = [start(0)]
        for i, (r, c) in enumerate(tasks):
            if i + 1 < len(tasks):
                copies.append(start(i + 1))
            if c == 0:
                for bb in range(_BINS):
                    hist[pl.ds(bb * _LANES, _LANES)] = zeros
            copies[i].wait()
            bref = buf.at[i % 2]

            @plsc.parallel_loop(0, _VREGS_PER_CHUNK, step=1, unroll=_UNROLL)
            def _(j):
                rr = lax.shift_right_logical(j, 5)
                cc = lax.shift_left(jnp.bitwise_and(j, 31), 4)
                v = bref[rr, pl.ds(cc, _LANES)]
                bits = plsc.bitcast(v + 1.0, jnp.int32)
                b16x = jnp.bitwise_and(
                    lax.shift_right_logical(bits, 15), 0xF0)
                idx = b16x + lane
                plsc.addupdate_scatter(hist, [idx], ones)

            if c == _NCHUNK - 1:
                tot = zeros
                lane16 = lane * 16
                for l in range(_LANES):
                    tot = tot + plsc.load_gather(hist, [lane16 + l])
                outv[r] = tot * (1.0 / _PIX)

        pltpu.sync_copy(outv, out_hbm.at[wid])

    return body(x_flat)


def _tc_hist(xr):
    k = _ROWS_TC

    def hbody(x_ref, o_ref):
        v = x_ref[0]
        bits = (v + 1.0).view(jnp.int32)
        bn = jnp.bitwise_and(lax.shift_right_logical(bits, 19), 0xF)
        bn16 = bn.astype(jnp.int16)
        cols = []
        for i in range(_BINS):
            m = (bn16 == jnp.int16(i)).astype(jnp.int16)
            n = m.shape[0]
            while n > 8:
                n //= 2
                m = m[:n] + m[n:]
            cols.append(jnp.sum(m.astype(jnp.float32)))
        row = jnp.concatenate(
            [jnp.stack(cols) * (1.0 / _PIX),
             jnp.zeros((128 - _BINS,), jnp.float32)])
        o_ref[...] = jnp.broadcast_to(row.reshape(1, 1, 128), (1, 8, 128))

    return pl.pallas_call(
        hbody,
        grid=(k,),
        in_specs=[pl.BlockSpec((1, _IMG, _IMG),
                               lambda i: (i + _ROWS_SC, 0, 0))],
        out_specs=pl.BlockSpec((1, 8, 128), lambda i: (i, 0, 0)),
        out_shape=jax.ShapeDtypeStruct((k, 8, 128), jnp.float32),
    )(xr)


def _fc(h, W, b):
    def fc_body(h_ref, w_ref, b_ref, o_ref):
        acc = jnp.dot(h_ref[...], w_ref[...],
                      preferred_element_type=jnp.float32)
        o_ref[...] = jnp.maximum(acc + b_ref[...], 0.0)

    return pl.pallas_call(
        fc_body,
        out_shape=jax.ShapeDtypeStruct((32, 64), jnp.float32),
    )(h, W, b.reshape(1, 64))


def kernel(x, W, b):
    x_flat = x.reshape(_ROWS, _IMG, _IMG)
    sc_means = _sc_hist(x_flat)
    tc_out = _tc_hist(x_flat)
    h = jnp.concatenate(
        [sc_means.reshape(_ROWS_SC, _BINS), tc_out[:, 0, :_BINS]],
        axis=0).reshape(32, 48)
    return _fc(h, W, b)


# speedup vs baseline: 1.3594x; 1.0087x over previous
import functools

import jax
import jax.numpy as jnp
from jax import lax
from jax.experimental import pallas as pl
from jax.experimental.pallas import tpu as pltpu
from jax.experimental.pallas import tpu_sc as plsc

_BINS = 16
_LANES = 16
_ROWS = 96
_IMG = 512
_PIX = _IMG * _IMG
_CHROWS = 64
_CHUNK = _CHROWS * _IMG
_NCHUNK = _PIX // _CHUNK
_UNROLL = 16
_VREGS_PER_CHUNK = _CHUNK // _LANES

_INFO = plsc.get_sparse_core_info()
_NC = _INFO.num_cores
_NS = _INFO.num_subcores
_NW = _NC * _NS
_ROWS_SC = 64
_ROWS_TC = _ROWS - _ROWS_SC
_ROWS_PER_W = _ROWS_SC // _NW


def _sc_hist(x_flat):
    mesh = plsc.VectorSubcoreMesh(core_axis_name="c", subcore_axis_name="s")

    @functools.partial(
        pl.kernel,
        out_type=jax.ShapeDtypeStruct((_NW, _ROWS_PER_W, _BINS), jnp.float32),
        mesh=mesh,
        compiler_params=pltpu.CompilerParams(needs_layout_passes=False),
        scratch_types=[
            pltpu.VMEM((2, _CHROWS, _IMG), jnp.float32),
            pltpu.VMEM((_BINS * _LANES,), jnp.float32),
            pltpu.VMEM((_ROWS_PER_W, _BINS), jnp.float32),
            pltpu.SemaphoreType.DMA,
            pltpu.SemaphoreType.DMA,
        ],
    )
    def body(x_hbm, out_hbm, buf, hist, outv, sem0, sem1):
        wid = lax.axis_index("s") * _NC + lax.axis_index("c")
        row0 = wid * _ROWS_PER_W
        sems = (sem0, sem1)
        lane = lax.iota(jnp.int32, 16)
        ones = jnp.ones((16,), jnp.float32)
        zeros = jnp.zeros((16,), jnp.float32)

        tasks = [(r, c) for r in range(_ROWS_PER_W) for c in range(_NCHUNK)]

        def start(i):
            r, c = tasks[i]
            return pltpu.async_copy(
                x_hbm.at[row0 + r, pl.ds(c * _CHROWS, _CHROWS), :],
                buf.at[i % 2],
                sems[i % 2],
            )

        copies = [start(0)]
        for i, (r, c) in enumerate(tasks):
            if i + 1 < len(tasks):
                copies.append(start(i + 1))
            if c == 0:
                for bb in range(_BINS):
                    hist[pl.ds(bb * _LANES, _LANES)] = zeros
            copies[i].wait()
            bref = buf.at[i % 2]

            @plsc.parallel_loop(0, _VREGS_PER_CHUNK, step=1, unroll=_UNROLL)
            def _(j):
                rr = lax.shift_right_logical(j, 5)
                cc = lax.shift_left(jnp.bitwise_and(j, 31), 4)
                v = bref[rr, pl.ds(cc, _LANES)]
                bits = plsc.bitcast(v + 1.0, jnp.int32)
                b16x = jnp.bitwise_and(
                    lax.shift_right_logical(bits, 15), 0xF0)
                idx = b16x + lane
                plsc.addupdate_scatter(hist, [idx], ones)

            if c == _NCHUNK - 1:
                tot = zeros
                lane16 = lane * 16
                for l in range(_LANES):
                    tot = tot + plsc.load_gather(hist, [lane16 + l])
                outv[r] = tot * (1.0 / _PIX)

        pltpu.sync_copy(outv, out_hbm.at[wid])

    return body(x_flat)


def _tc_hist(xr):
    k = _ROWS_TC

    def hbody(x_ref, o_ref):
        v = x_ref[0]
        bits = (v + 1.0).view(jnp.int32)
        bn = jnp.bitwise_and(lax.shift_right_logical(bits, 19), 0xF)
        bn16 = bn.astype(jnp.int16)
        cols = []
        for i in range(_BINS):
            m = (bn16 == jnp.int16(i)).astype(jnp.int16)
            n = m.shape[0]
            while n > 8:
                n //= 2
                m = m[:n] + m[n:]
            cols.append(jnp.sum(m.astype(jnp.float32)))
        row = jnp.concatenate(
            [jnp.stack(cols) * (1.0 / _PIX),
             jnp.zeros((128 - _BINS,), jnp.float32)])
        o_ref[...] = jnp.broadcast_to(row.reshape(1, 1, 128), (1, 8, 128))

    return pl.pallas_call(
        hbody,
        grid=(k,),
        in_specs=[pl.BlockSpec((1, _IMG, _IMG),
                               lambda i: (i + _ROWS_SC, 0, 0))],
        out_specs=pl.BlockSpec((1, 8, 128), lambda i: (i, 0, 0)),
        out_shape=jax.ShapeDtypeStruct((k, 8, 128), jnp.float32),
    )(xr)


def _fc(h, W, b):
    def fc_body(h_ref, w_ref, b_ref, o_ref):
        acc = jnp.dot(h_ref[...], w_ref[...],
                      preferred_element_type=jnp.float32)
        o_ref[...] = jnp.maximum(acc + b_ref[...], 0.0)

    return pl.pallas_call(
        fc_body,
        out_shape=jax.ShapeDtypeStruct((32, 64), jnp.float32),
    )(h, W, b.reshape(1, 64))


def kernel(x, W, b):
    x_flat = x.reshape(_ROWS, _IMG, _IMG)
    sc_means = _sc_hist(x_flat)
    tc_out = _tc_hist(x_flat)
    h = jnp.concatenate(
        [sc_means.reshape(_ROWS_SC, _BINS), tc_out[:, 0, :_BINS]],
        axis=0).reshape(32, 48)
    return _fc(h, W, b)
